# Initial kernel scaffold; baseline (speedup 1.0000x reference)
import jax
import jax.numpy as jnp
from jax.experimental import pallas as pl


def kernel(logits_student, logits_teacher, edge_index, homophily):
    raise NotImplementedError("write your pallas kernel here")


# speedup vs baseline: 4.6766x; 4.6766x over previous
import functools

import jax
import jax.numpy as jnp
from jax import lax
from jax.experimental import pallas as pl
from jax.experimental.pallas import tpu as pltpu
from jax.experimental.pallas import tpu_sc as plsc

N = 10000
E = 320000
C = 128
TEMP = 4.0
HIGH_FREQ_SCALE = 2.0

NUM_CORES = 2
NUM_SUBCORES = 16
CHUNK = 128
EDGES_PER_TILE = 20096
E_PAD = EDGES_PER_TILE * NUM_SUBCORES
CHUNKS_PER_TILE = EDGES_PER_TILE // CHUNK
N_PAD = 10240
ROWS_PER_TILE = N_PAD // NUM_SUBCORES


def _sc_segment_sums(zt, row, col2):
    mesh = plsc.VectorSubcoreMesh(core_axis_name="c", subcore_axis_name="s")

    @functools.partial(
        pl.kernel,
        mesh=mesh,
        out_type=[
            jax.ShapeDtypeStruct((NUM_CORES, N_PAD, C), jnp.float32),
            jax.ShapeDtypeStruct((N_PAD,), jnp.float32),
        ],
        scratch_types=[
            pltpu.VMEM((CHUNK,), jnp.int32),
            pltpu.VMEM((CHUNK,), jnp.int32),
            pltpu.VMEM((CHUNK, C), jnp.float32),
            pltpu.VMEM((ROWS_PER_TILE,), jnp.float32),
            pltpu.VMEM((CHUNK,), jnp.float32),
            pltpu.VMEM_SHARED((N_PAD, C), jnp.float32),
            pltpu.VMEM_SHARED((N_PAD,), jnp.float32),
            pltpu.SemaphoreType.DMA,
        ],
    )
    def k(zt_hbm, row_hbm, col2_hbm, sums_hbm, deg_hbm,
          idx_row, idx_col, rows_v, zdeg, ones_v, acc, dacc, sem):
        c = lax.axis_index("c")
        s = lax.axis_index("s")
        zero16 = jnp.zeros((16,), jnp.float32)

        def zrow_body(i, carry):
            for kk in range(C // 16):
                rows_v[i, pl.ds(kk * 16, 16)] = zero16
            return carry
        lax.fori_loop(0, CHUNK, zrow_body, 0)

        def zdeg_body(i, carry):
            zdeg[pl.ds(i * 16, 16)] = zero16
            return carry
        lax.fori_loop(0, ROWS_PER_TILE // 16, zdeg_body, 0)

        def ones_body(i, carry):
            ones_v[pl.ds(i * 16, 16)] = zero16 + 1.0
            return carry
        lax.fori_loop(0, CHUNK // 16, ones_body, 0)

        base_rows = s * ROWS_PER_TILE
        for j in range(ROWS_PER_TILE // CHUNK):
            pltpu.sync_copy(rows_v, acc.at[pl.ds(base_rows + j * CHUNK, CHUNK)])
        pltpu.sync_copy(zdeg, dacc.at[pl.ds(base_rows, ROWS_PER_TILE)])
        plsc.subcore_barrier()

        edge_base = s * EDGES_PER_TILE

        def body(i, carry):
            b = edge_base + i * CHUNK
            pltpu.sync_copy(row_hbm.at[pl.ds(b, CHUNK)], idx_row)
            pltpu.sync_copy(col2_hbm.at[c, pl.ds(b, CHUNK)], idx_col)
            pltpu.async_copy(zt_hbm.at[idx_col], rows_v, sem).wait()
            pltpu.sync_copy(rows_v, acc.at[idx_row], add=True)
            pltpu.sync_copy(ones_v, dacc.at[idx_row], add=True)
            return carry
        lax.fori_loop(0, CHUNKS_PER_TILE, body, 0)

        plsc.subcore_barrier()

        for j in range(ROWS_PER_TILE // CHUNK):
            r = base_rows + j * CHUNK
            pltpu.sync_copy(acc.at[pl.ds(r, CHUNK)], sums_hbm.at[c, pl.ds(r, CHUNK)])

        @pl.when(c == 0)
        def _():
            pltpu.sync_copy(dacc.at[pl.ds(base_rows, ROWS_PER_TILE)],
                            deg_hbm.at[pl.ds(base_rows, ROWS_PER_TILE)])

    return k(zt, row, col2)


def _tc_loss_body(s_ref, t_ref, ss_ref, ts_ref, deg_ref, h_ref, out_ref):
    pi = pl.program_id(0)
    s = s_ref[...]
    t = t_ref[...]
    d = deg_ref[...]
    d = jnp.where(d == 0.0, 1.0, d)
    s_low = ss_ref[...] / d
    t_low = ts_ref[...] / d

    inv_t = jnp.float32(1.0 / TEMP)
    xs = s_low * inv_t
    xt = t_low * inv_t
    ms = jnp.max(xs, axis=1, keepdims=True)
    log_ps = xs - ms - jnp.log(jnp.sum(jnp.exp(xs - ms), axis=1, keepdims=True))
    mt = jnp.max(xt, axis=1, keepdims=True)
    log_pt = xt - mt - jnp.log(jnp.sum(jnp.exp(xt - mt), axis=1, keepdims=True))
    p_t = jnp.exp(log_pt)
    loss_low = jnp.sum(p_t * (log_pt - log_ps), axis=1, keepdims=True) * jnp.float32(TEMP * TEMP)

    diff = (s - t) - (s_low - t_low)
    loss_high = jnp.sum(diff * diff, axis=1, keepdims=True) * jnp.float32(HIGH_FREQ_SCALE / C)

    h = h_ref[...]
    wl = h * loss_low + (1.0 - h) * loss_high
    part = jnp.sum(wl) * jnp.float32(1.0 / N)

    @pl.when(pi == 0)
    def _():
        out_ref[0, 0] = part

    @pl.when(pi != 0)
    def _():
        out_ref[0, 0] += part


def _tc_loss(s, t, ssum, tsum, deg, h):
    blocks = 5
    br = N // blocks
    grid = (blocks,)
    row_spec = pl.BlockSpec((br, C), lambda i: (i, 0))
    col_spec = pl.BlockSpec((br, 1), lambda i: (i, 0))
    out = pl.pallas_call(
        _tc_loss_body,
        grid=grid,
        in_specs=[row_spec, row_spec, row_spec, row_spec, col_spec, col_spec],
        out_specs=pl.BlockSpec(memory_space=pltpu.MemorySpace.SMEM),
        out_shape=jax.ShapeDtypeStruct((1, 1), jnp.float32),
    )(s, t, ssum, tsum, deg, h)
    return jnp.reshape(out, ())


def kernel(logits_student, logits_teacher, edge_index, homophily):
    row = edge_index[0]
    col = edge_index[1]
    pad = E_PAD - E
    row_p = jnp.concatenate([row, jnp.full((pad,), N, jnp.int32)])
    col_p = jnp.concatenate([col, jnp.zeros((pad,), jnp.int32)])
    col2 = jnp.stack([col_p, col_p + N])
    zt = jnp.concatenate([logits_student, logits_teacher], axis=0)

    sums, deg = _sc_segment_sums(zt, row_p, col2)
    ssum = sums[0, :N]
    tsum = sums[1, :N]
    deg = deg[:N].reshape(N, 1)
    h = homophily.reshape(N, 1)
    return _tc_loss(logits_student, logits_teacher, ssum, tsum, deg, h)
